# baseline (speedup 1.0000x reference)
import functools

import jax
import jax.numpy as jnp
from jax import lax
from jax.experimental import pallas as pl
from jax.experimental.pallas import tpu as pltpu
from jax.experimental.pallas import tpu_sc as plsc

MAXP = 64
K = 9
D = 256
NCH = D // 16
TAU = 0.07
NC, NS, L = 2, 16, 16
NW = NC * NS
GROUP = 32
T_SC = 1024
BT = 1024
PROW = 17


def _rsqrt16(x):
    i = plsc.bitcast(x, jnp.int32)
    y = plsc.bitcast(jnp.int32(0x5F3759DF) - (i >> 1), jnp.float32)
    for _ in range(3):
        y = y * (1.5 - 0.5 * x * y * y)
    return y


def _make_sc_call(total_t, t_sc):
    off = total_t - t_sc
    TPW = t_sc // NW
    NG = TPW // GROUP
    mesh = plsc.VectorSubcoreMesh(
        core_axis_name="c", subcore_axis_name="s",
        num_cores=NC, num_subcores=NS)

    @functools.partial(
        pl.kernel,
        out_type=jax.ShapeDtypeStruct((K * t_sc,), jnp.float32),
        mesh=mesh,
        scratch_types=[
            pltpu.VMEM((GROUP, D), jnp.float32),
            pltpu.VMEM((GROUP, D), jnp.float32),
            pltpu.VMEM((L * D,), jnp.float32),
            pltpu.VMEM((L * L * PROW,), jnp.float32),
            pltpu.VMEM((L * GROUP,), jnp.float32),
            pltpu.SemaphoreType.DMA,
            pltpu.SemaphoreType.DMA,
        ],
        compiler_params=pltpu.CompilerParams(needs_layout_passes=False),
    )
    def sc_call(z_hbm, p_hbm, out_hbm, zb0, zb1, pbuf, partbuf, obuf,
                sem0, sem1):
        wid = lax.axis_index("s") * NC + lax.axis_index("c")
        lane = lax.iota(jnp.int32, L)
        lane_p = lane * PROW

        pltpu.sync_copy(p_hbm.at[pl.ds(0, L * D)], pbuf)
        psvec = jnp.zeros((L,), jnp.float32)
        for k in range(K):
            vacc = jnp.zeros((L,), jnp.float32)
            for c in range(NCH):
                v = pbuf[pl.ds(k * D + c * L, L)]
                vacc = vacc + v * v
            nsq = lax.reduce_sum_p.bind(vacc, axes=(0,))
            sk = _rsqrt16(jnp.maximum(
                jnp.broadcast_to(nsq, (L,)), 1e-24)) * (1.0 / TAU)
            psvec = jnp.where(lane == k, sk, psvec)

        zbufs = (zb0, zb1)
        sems = (sem0, sem1)
        tbase = off + wid * TPW

        def _start(g, slot):
            return pltpu.async_copy(
                z_hbm.at[pl.ds(tbase + g * GROUP, GROUP)],
                zbufs[slot], sems[slot])

        cps = [_start(0, 0), None]
        for g in range(NG):
            cur = g & 1
            if g + 1 < NG:
                cps[1 - cur] = _start(g + 1, 1 - cur)
            cps[cur].wait()
            zb = zbufs[cur]

            for blk in range(GROUP // L):
                for pair in range(L // 2):
                    tl0, tl1 = 2 * pair, 2 * pair + 1
                    r0 = jnp.full((L,), blk * L + tl0, jnp.int32)
                    r1 = jnp.full((L,), blk * L + tl1, jnp.int32)

                    def _chunk(ci, accs, zb=zb, r0=r0, r1=r1):
                        cols = [lane + (2 * ci + u) * L for u in range(2)]
                        zvs = [[plsc.load_gather(zb, [r, col])
                                for r in (r0, r1)] for col in cols]
                        pvs = [[plsc.load_gather(pbuf, [col + k * D])
                                for k in range(K)] for col in cols]
                        n = [list(a) for a in accs]
                        for u in range(2):
                            for k in range(K):
                                n[0][k] = n[0][k] + zvs[u][0] * pvs[u][k]
                                n[1][k] = n[1][k] + zvs[u][1] * pvs[u][k]
                            n[0][K] = n[0][K] + zvs[u][0] * zvs[u][0]
                            n[1][K] = n[1][K] + zvs[u][1] * zvs[u][1]
                        return (tuple(n[0]), tuple(n[1]))

                    zero = jnp.zeros((L,), jnp.float32)
                    init = (tuple(zero for _ in range(K + 1)),) * 2
                    accs = lax.fori_loop(0, NCH // 2, _chunk, init)

                    sks = [[lax.reduce_sum_p.bind(a, axes=(0,))
                            for a in acc] for acc in accs]
                    for tl, sk in ((tl0, sks[0]), (tl1, sks[1])):
                        s = jnp.zeros((L,), jnp.float32)
                        for k in range(K + 1):
                            s = jnp.where(
                                lane == k,
                                jnp.broadcast_to(sk[k], (L,)), s)
                        nsq = jnp.broadcast_to(sk[K], (L,))
                        zinv = _rsqrt16(jnp.maximum(nsq, 1e-24))
                        tok = blk * L + tl
                        plsc.store_scatter(
                            obuf, [lane * GROUP + tok], s * zinv * psvec)

            col0 = wid * TPW + g * GROUP
            for k in range(K):
                pltpu.sync_copy(
                    obuf.at[pl.ds(k * GROUP, GROUP)],
                    out_hbm.at[pl.ds(k * t_sc + col0, GROUP)])

    return sc_call


def _tc_body(p_ref, z_ref, o_ref):
    p = p_ref[:K, :]
    pn = p * lax.rsqrt(
        jnp.maximum(jnp.sum(p * p, axis=-1, keepdims=True), 1e-24))
    z = z_ref[...]
    zinv = lax.rsqrt(
        jnp.maximum(jnp.sum(z * z, axis=-1), 1e-24)) * (1.0 / TAU)
    d = lax.dot_general(
        pn, z, (((1,), (1,)), ((), ())),
        preferred_element_type=jnp.float32)
    o_ref[...] = d * zinv[None, :]


def _tc_call(z2d, prototypes, t_tc):
    return pl.pallas_call(
        _tc_body,
        grid=(t_tc // BT,),
        in_specs=[
            pl.BlockSpec((MAXP, D), lambda i: (0, 0)),
            pl.BlockSpec((BT, D), lambda i: (i, 0)),
        ],
        out_specs=pl.BlockSpec((K, BT), lambda i: (0, i)),
        out_shape=jax.ShapeDtypeStruct((K, t_tc), jnp.float32),
    )(prototypes, z2d)


def kernel(hidden_z, prototypes):
    B, S, _ = hidden_z.shape
    T = B * S
    t_tc = T - T_SC
    z2d = hidden_z.reshape(T, D)
    pf = prototypes.reshape(MAXP * D)
    out_sc = _make_sc_call(T, T_SC)(z2d, pf)
    out_tc = _tc_call(z2d, prototypes, t_tc)
    out = jnp.concatenate(
        [out_tc, out_sc.reshape(K, T_SC)], axis=1)
    return out.reshape(K, B, S).transpose(1, 2, 0)


# speedup vs baseline: 1.2954x; 1.2954x over previous
import functools

import jax
import jax.numpy as jnp
from jax import lax
from jax.experimental import pallas as pl
from jax.experimental.pallas import tpu as pltpu
from jax.experimental.pallas import tpu_sc as plsc

MAXP = 64
K = 9
D = 256
NCH = D // 16
TAU = 0.07
NC, NS, L = 2, 16, 16
NW = NC * NS
GROUP = 16
T_SC = 512
BT = 4608
PROW = 17


def _rsqrt16(x):
    i = plsc.bitcast(x, jnp.int32)
    y = plsc.bitcast(jnp.int32(0x5F3759DF) - (i >> 1), jnp.float32)
    for _ in range(3):
        y = y * (1.5 - 0.5 * x * y * y)
    return y


def _make_sc_call(total_t, t_sc):
    off = total_t - t_sc
    TPW = t_sc // NW
    NG = TPW // GROUP
    mesh = plsc.VectorSubcoreMesh(
        core_axis_name="c", subcore_axis_name="s",
        num_cores=NC, num_subcores=NS)

    @functools.partial(
        pl.kernel,
        out_type=jax.ShapeDtypeStruct((K * t_sc,), jnp.float32),
        mesh=mesh,
        scratch_types=[
            pltpu.VMEM((GROUP, D), jnp.float32),
            pltpu.VMEM((GROUP, D), jnp.float32),
            pltpu.VMEM((L * D,), jnp.float32),
            pltpu.VMEM((L * L * PROW,), jnp.float32),
            pltpu.VMEM((L * GROUP,), jnp.float32),
            pltpu.SemaphoreType.DMA,
            pltpu.SemaphoreType.DMA,
        ],
        compiler_params=pltpu.CompilerParams(needs_layout_passes=False),
    )
    def sc_call(z_hbm, p_hbm, out_hbm, zb0, zb1, pbuf, partbuf, obuf,
                sem0, sem1):
        wid = lax.axis_index("s") * NC + lax.axis_index("c")
        lane = lax.iota(jnp.int32, L)
        lane_p = lane * PROW

        pltpu.sync_copy(p_hbm.at[pl.ds(0, L * D)], pbuf)
        psvec = jnp.zeros((L,), jnp.float32)
        for k in range(K):
            vacc = jnp.zeros((L,), jnp.float32)
            for c in range(NCH):
                v = pbuf[pl.ds(k * D + c * L, L)]
                vacc = vacc + v * v
            nsq = lax.reduce_sum_p.bind(vacc, axes=(0,))
            sk = _rsqrt16(jnp.maximum(
                jnp.broadcast_to(nsq, (L,)), 1e-24)) * (1.0 / TAU)
            psvec = jnp.where(lane == k, sk, psvec)

        zbufs = (zb0, zb1)
        sems = (sem0, sem1)
        tbase = off + wid * TPW

        def _start(g, slot):
            return pltpu.async_copy(
                z_hbm.at[pl.ds(tbase + g * GROUP, GROUP)],
                zbufs[slot], sems[slot])

        cps = [_start(0, 0), None]
        for g in range(NG):
            cur = g & 1
            if g + 1 < NG:
                cps[1 - cur] = _start(g + 1, 1 - cur)
            cps[cur].wait()
            zb = zbufs[cur]

            for blk in range(GROUP // L):
                for pair in range(L // 2):
                    tl0, tl1 = 2 * pair, 2 * pair + 1
                    r0 = jnp.full((L,), blk * L + tl0, jnp.int32)
                    r1 = jnp.full((L,), blk * L + tl1, jnp.int32)

                    def _chunk(ci, accs, zb=zb, r0=r0, r1=r1):
                        cols = [lane + (2 * ci + u) * L for u in range(2)]
                        zvs = [[plsc.load_gather(zb, [r, col])
                                for r in (r0, r1)] for col in cols]
                        pvs = [[plsc.load_gather(pbuf, [col + k * D])
                                for k in range(K)] for col in cols]
                        n = [list(a) for a in accs]
                        for u in range(2):
                            for k in range(K):
                                n[0][k] = n[0][k] + zvs[u][0] * pvs[u][k]
                                n[1][k] = n[1][k] + zvs[u][1] * pvs[u][k]
                            n[0][K] = n[0][K] + zvs[u][0] * zvs[u][0]
                            n[1][K] = n[1][K] + zvs[u][1] * zvs[u][1]
                        return (tuple(n[0]), tuple(n[1]))

                    zero = jnp.zeros((L,), jnp.float32)
                    init = (tuple(zero for _ in range(K + 1)),) * 2
                    accs = lax.fori_loop(0, NCH // 2, _chunk, init)

                    sks = [[lax.reduce_sum_p.bind(a, axes=(0,))
                            for a in acc] for acc in accs]
                    for tl, sk in ((tl0, sks[0]), (tl1, sks[1])):
                        s = jnp.zeros((L,), jnp.float32)
                        for k in range(K + 1):
                            s = jnp.where(
                                lane == k,
                                jnp.broadcast_to(sk[k], (L,)), s)
                        nsq = jnp.broadcast_to(sk[K], (L,))
                        zinv = _rsqrt16(jnp.maximum(nsq, 1e-24))
                        tok = blk * L + tl
                        plsc.store_scatter(
                            obuf, [lane * GROUP + tok], s * zinv * psvec)

            col0 = wid * TPW + g * GROUP
            for k in range(K):
                pltpu.sync_copy(
                    obuf.at[pl.ds(k * GROUP, GROUP)],
                    out_hbm.at[pl.ds(k * t_sc + col0, GROUP)])

    return sc_call


def _tc_body(p_ref, z_ref, o_ref):
    p = p_ref[:K, :]
    pn = p * lax.rsqrt(
        jnp.maximum(jnp.sum(p * p, axis=-1, keepdims=True), 1e-24))
    z = z_ref[...]
    zinv = lax.rsqrt(
        jnp.maximum(jnp.sum(z * z, axis=-1), 1e-24)) * (1.0 / TAU)
    d = lax.dot_general(
        pn, z, (((1,), (1,)), ((), ())),
        preferred_element_type=jnp.float32)
    o_ref[...] = d * zinv[None, :]


def _tc_call(z2d, prototypes, t_tc):
    return pl.pallas_call(
        _tc_body,
        grid=(t_tc // BT,),
        in_specs=[
            pl.BlockSpec((MAXP, D), lambda i: (0, 0)),
            pl.BlockSpec((BT, D), lambda i: (i, 0)),
        ],
        out_specs=pl.BlockSpec((K, BT), lambda i: (0, i)),
        out_shape=jax.ShapeDtypeStruct((K, t_tc), jnp.float32),
    )(prototypes, z2d)


def kernel(hidden_z, prototypes):
    B, S, _ = hidden_z.shape
    T = B * S
    t_tc = T - T_SC
    z2d = hidden_z.reshape(T, D)
    pf = prototypes.reshape(MAXP * D)
    out_sc = _make_sc_call(T, T_SC)(z2d, pf)
    out_tc = _tc_call(z2d, prototypes, t_tc)
    out = jnp.concatenate(
        [out_tc, out_sc.reshape(K, T_SC)], axis=1)
    return out.reshape(K, B, S).transpose(1, 2, 0)
